# Initial kernel scaffold; baseline (speedup 1.0000x reference)
import jax
import jax.numpy as jnp
from jax.experimental import pallas as pl


def kernel(source, target):
    raise NotImplementedError("write your pallas kernel here")


# speedup vs baseline: 1.9846x; 1.9846x over previous
import jax
import jax.numpy as jnp
from jax.experimental import pallas as pl
from jax.experimental.pallas import tpu as pltpu

_VMEM_LIMIT_BYTES = 48 * 1024 * 1024
_KERNEL_NUM = 5
_KERNEL_MUL = 2.0


def _stats_body(x_ref, csum_ref, norms_ref, ssq_ref):
    @pl.when(pl.program_id(0) == 0)
    def _init():
        csum_ref[...] = jnp.zeros_like(csum_ref)
        ssq_ref[...] = jnp.zeros_like(ssq_ref)

    x = x_ref[...]
    sq = x * x
    norms = jnp.sum(sq, axis=1, keepdims=True)
    csum_ref[...] = csum_ref[...] + jnp.sum(x, axis=0, keepdims=True)
    norms_ref[...] = norms
    ssq_ref[...] = ssq_ref[...] + jnp.sum(norms)


def _pair_body(ii_ref, jj_ref, coef_ref, sc_ref, m_ref,
               x_ref, y_ref, nr_ref, nc_ref, out_ref):
    s = pl.program_id(0)

    @pl.when(s == 0)
    def _init():
        out_ref[...] = jnp.zeros_like(out_ref)

    g = jax.lax.dot_general(
        x_ref[...], y_ref[...] * sc_ref[0], (((1,), (1,)), ((), ())),
        preferred_element_type=jnp.float32)

    z = g + nr_ref[...] * m_ref[0] + nc_ref[...] * m_ref[0]

    e = jnp.exp2(z)
    e2 = e * e
    e4 = e2 * e2
    e8 = e4 * e4
    e16 = e8 * e8
    ksum = ((e + e2) + (e4 + e8)) + e16

    out_ref[...] = out_ref[...] + coef_ref[s] * jnp.sum(ksum)


def _stats(x, tile):
    b, d = x.shape
    return pl.pallas_call(
        _stats_body,
        out_shape=(jax.ShapeDtypeStruct((1, d), jnp.float32),
                   jax.ShapeDtypeStruct((b, 1), jnp.float32),
                   jax.ShapeDtypeStruct((8, 128), jnp.float32)),
        grid=(b // tile,),
        in_specs=[pl.BlockSpec((tile, d), lambda i: (i, 0))],
        out_specs=(pl.BlockSpec((1, d), lambda i: (0, 0)),
                   pl.BlockSpec((tile, 1), lambda i: (i, 0)),
                   pl.BlockSpec((8, 128), lambda i: (0, 0))),
        compiler_params=pltpu.CompilerParams(
            dimension_semantics=("arbitrary",),
            vmem_limit_bytes=_VMEM_LIMIT_BYTES),
    )(x)


def _pair_sweep(x, y, nr, nc, ii_list, jj_list, coef_list, sc, m, tile):
    d = x.shape[1]
    ns = len(ii_list)
    ii = jnp.asarray(ii_list, dtype=jnp.int32)
    jj = jnp.asarray(jj_list, dtype=jnp.int32)
    coef = jnp.asarray(coef_list, dtype=jnp.float32)
    partials = pl.pallas_call(
        _pair_body,
        out_shape=jax.ShapeDtypeStruct((8, 128), jnp.float32),
        grid_spec=pltpu.PrefetchScalarGridSpec(
            num_scalar_prefetch=5,
            grid=(ns,),
            in_specs=[
                pl.BlockSpec((tile, d),
                             lambda s, ir, jr, cf, sr, mr: (ir[s], 0)),
                pl.BlockSpec((tile, d),
                             lambda s, ir, jr, cf, sr, mr: (jr[s], 0)),
                pl.BlockSpec((tile, 1),
                             lambda s, ir, jr, cf, sr, mr: (ir[s], 0)),
                pl.BlockSpec((1, tile),
                             lambda s, ir, jr, cf, sr, mr: (0, jr[s])),
            ],
            out_specs=pl.BlockSpec((8, 128),
                                   lambda s, ir, jr, cf, sr, mr: (0, 0)),
        ),
        compiler_params=pltpu.CompilerParams(
            dimension_semantics=("arbitrary",),
            vmem_limit_bytes=_VMEM_LIMIT_BYTES),
    )(ii, jj, coef, sc, m, x, y, nr, nc)
    return partials[0, 0]


def _upper_tri(nt, w2):
    ii, jj, coef = [], [], []
    for a in range(nt):
        for b in range(a, nt):
            ii.append(a)
            jj.append(b)
            coef.append((2.0 if b > a else 1.0) * w2)
    return ii, jj, coef


def _rect(nt_r, nt_c, w2):
    ii, jj, coef = [], [], []
    for a in range(nt_r):
        for b in range(nt_c):
            ii.append(a)
            jj.append(b)
            coef.append(w2)
    return ii, jj, coef


def _mmd_loss(source, target, tile):
    b_src, d = source.shape
    b_tgt = target.shape[0]
    n = b_src + b_tgt
    assert b_src % tile == 0 and b_tgt % tile == 0 and d % 128 == 0
    nt_s = b_src // tile
    nt_t = b_tgt // tile

    source = source.astype(jnp.float32)
    target = target.astype(jnp.float32)

    csum_s, norms_s, ssq_s = _stats(source, tile)
    csum_t, norms_t, ssq_t = _stats(target, tile)

    ssq = ssq_s[0, 0] + ssq_t[0, 0]
    csum = csum_s + csum_t
    sum_l2 = 2.0 * float(n) * ssq - 2.0 * jnp.sum(csum * csum)
    bandwidth = sum_l2 / float(n * n - n)
    bandwidth = bandwidth / (_KERNEL_MUL ** (_KERNEL_NUM // 2))
    inv_bw_largest = 1.0 / (bandwidth * _KERNEL_MUL ** (_KERNEL_NUM - 1))

    log2e = 1.4426950408889634
    sc = (2.0 * log2e * inv_bw_largest).reshape(1).astype(jnp.float32)
    m = (-log2e * inv_bw_largest).reshape(1).astype(jnp.float32)

    nc_s = norms_s.reshape(1, b_src)
    nc_t = norms_t.reshape(1, b_tgt)

    w_ss = 1.0 / (float(b_src) * b_src)
    w_tt = 1.0 / (float(b_tgt) * b_tgt)
    w_st = -2.0 / (float(b_src) * b_tgt)

    p_ss = _pair_sweep(source, source, norms_s, nc_s,
                       *_upper_tri(nt_s, w_ss), sc, m, tile)
    p_tt = _pair_sweep(target, target, norms_t, nc_t,
                       *_upper_tri(nt_t, w_tt), sc, m, tile)
    p_st = _pair_sweep(source, target, norms_s, nc_t,
                       *_rect(nt_s, nt_t, w_st), sc, m, tile)

    return p_ss + p_tt + p_st


def kernel(source, target):
    return _mmd_loss(source, target, tile=1024)


# speedup vs baseline: 2.0297x; 1.0227x over previous
import jax
import jax.numpy as jnp
from jax.experimental import pallas as pl
from jax.experimental.pallas import tpu as pltpu

_VMEM_LIMIT_BYTES = 48 * 1024 * 1024
_KERNEL_NUM = 5
_KERNEL_MUL = 2.0


def _stats_body(x_ref, csum_ref, norms_ref, ssq_ref):
    @pl.when(pl.program_id(0) == 0)
    def _init():
        csum_ref[...] = jnp.zeros_like(csum_ref)
        ssq_ref[...] = jnp.zeros_like(ssq_ref)

    x = x_ref[...]
    sq = x * x
    norms = jnp.sum(sq, axis=1, keepdims=True)
    csum_ref[...] = csum_ref[...] + jnp.sum(x, axis=0, keepdims=True)
    norms_ref[...] = norms
    ssq_ref[...] = ssq_ref[...] + jnp.sum(norms)


def _pair_body(ii_ref, jj_ref, coef_ref, sc_ref, m_ref,
               x_ref, y_ref, nr_ref, nc_ref, out_ref):
    s = pl.program_id(0)

    @pl.when(s == 0)
    def _init():
        out_ref[...] = jnp.zeros_like(out_ref)

    g = jax.lax.dot_general(
        x_ref[...], y_ref[...] * sc_ref[0], (((1,), (1,)), ((), ())),
        preferred_element_type=jnp.float32)

    z = g + nr_ref[...] * m_ref[0] + nc_ref[...] * m_ref[0]

    e = jnp.exp2(z)
    e2 = e * e
    e4 = e2 * e2
    e8 = e4 * e4
    e16 = e8 * e8
    ksum = ((e + e2) + (e4 + e8)) + e16

    out_ref[...] = out_ref[...] + coef_ref[s] * jnp.sum(ksum)


def _stats(x, tile):
    b, d = x.shape
    return pl.pallas_call(
        _stats_body,
        out_shape=(jax.ShapeDtypeStruct((1, d), jnp.float32),
                   jax.ShapeDtypeStruct((b, 1), jnp.float32),
                   jax.ShapeDtypeStruct((8, 128), jnp.float32)),
        grid=(b // tile,),
        in_specs=[pl.BlockSpec((tile, d), lambda i: (i, 0))],
        out_specs=(pl.BlockSpec((1, d), lambda i: (0, 0)),
                   pl.BlockSpec((tile, 1), lambda i: (i, 0)),
                   pl.BlockSpec((8, 128), lambda i: (0, 0))),
        compiler_params=pltpu.CompilerParams(
            dimension_semantics=("arbitrary",),
            vmem_limit_bytes=_VMEM_LIMIT_BYTES),
    )(x)


def _pair_sweep(x, y, nr, nc, ii_list, jj_list, coef_list, sc, m,
                tile_r, tile_c):
    d = x.shape[1]
    ns = len(ii_list)
    ii = jnp.asarray(ii_list, dtype=jnp.int32)
    jj = jnp.asarray(jj_list, dtype=jnp.int32)
    coef = jnp.asarray(coef_list, dtype=jnp.float32)
    partials = pl.pallas_call(
        _pair_body,
        out_shape=jax.ShapeDtypeStruct((8, 128), jnp.float32),
        grid_spec=pltpu.PrefetchScalarGridSpec(
            num_scalar_prefetch=5,
            grid=(ns,),
            in_specs=[
                pl.BlockSpec((tile_r, d),
                             lambda s, ir, jr, cf, sr, mr: (ir[s], 0)),
                pl.BlockSpec((tile_c, d),
                             lambda s, ir, jr, cf, sr, mr: (jr[s], 0)),
                pl.BlockSpec((tile_r, 1),
                             lambda s, ir, jr, cf, sr, mr: (ir[s], 0)),
                pl.BlockSpec((1, tile_c),
                             lambda s, ir, jr, cf, sr, mr: (0, jr[s])),
            ],
            out_specs=pl.BlockSpec((8, 128),
                                   lambda s, ir, jr, cf, sr, mr: (0, 0)),
        ),
        compiler_params=pltpu.CompilerParams(
            dimension_semantics=("arbitrary",),
            vmem_limit_bytes=_VMEM_LIMIT_BYTES),
    )(ii, jj, coef, sc, m, x, y, nr, nc)
    return partials[0, 0]


def _upper_tri(nt, w2):
    ii, jj, coef = [], [], []
    for a in range(nt):
        for b in range(a, nt):
            ii.append(a)
            jj.append(b)
            coef.append((2.0 if b > a else 1.0) * w2)
    return ii, jj, coef


def _rect(nt_r, nt_c, w2):
    ii, jj, coef = [], [], []
    for a in range(nt_r):
        for b in range(nt_c):
            ii.append(a)
            jj.append(b)
            coef.append(w2)
    return ii, jj, coef


def _mmd_loss(source, target, tile):
    b_src, d = source.shape
    b_tgt = target.shape[0]
    n = b_src + b_tgt
    assert b_src % tile == 0 and b_tgt % tile == 0 and d % 128 == 0
    nt_s = b_src // tile
    nt_t = b_tgt // tile

    source = source.astype(jnp.float32)
    target = target.astype(jnp.float32)

    csum_s, norms_s, ssq_s = _stats(source, tile)
    csum_t, norms_t, ssq_t = _stats(target, tile)

    ssq = ssq_s[0, 0] + ssq_t[0, 0]
    csum = csum_s + csum_t
    sum_l2 = 2.0 * float(n) * ssq - 2.0 * jnp.sum(csum * csum)
    bandwidth = sum_l2 / float(n * n - n)
    bandwidth = bandwidth / (_KERNEL_MUL ** (_KERNEL_NUM // 2))
    inv_bw_largest = 1.0 / (bandwidth * _KERNEL_MUL ** (_KERNEL_NUM - 1))

    log2e = 1.4426950408889634
    sc = (2.0 * log2e * inv_bw_largest).reshape(1).astype(jnp.float32)
    m = (-log2e * inv_bw_largest).reshape(1).astype(jnp.float32)

    nc_s = norms_s.reshape(1, b_src)
    nc_t = norms_t.reshape(1, b_tgt)

    w_ss = 1.0 / (float(b_src) * b_src)
    w_tt = 1.0 / (float(b_tgt) * b_tgt)
    w_st = -2.0 / (float(b_src) * b_tgt)

    tile_st = 2 * tile if (b_src % (2 * tile) == 0 and
                           b_tgt % (2 * tile) == 0) else tile

    p_ss = _pair_sweep(source, source, norms_s, nc_s,
                       *_upper_tri(nt_s, w_ss), sc, m, tile, tile)
    p_tt = _pair_sweep(target, target, norms_t, nc_t,
                       *_upper_tri(nt_t, w_tt), sc, m, tile, tile)
    p_st = _pair_sweep(source, target, norms_s, nc_t,
                       *_rect(b_src // tile_st, b_tgt // tile_st, w_st),
                       sc, m, tile_st, tile_st)

    return p_ss + p_tt + p_st


def kernel(source, target):
    return _mmd_loss(source, target, tile=1024)


# speedup vs baseline: 2.0527x; 1.0113x over previous
import functools

import jax
import jax.numpy as jnp
from jax.experimental import pallas as pl
from jax.experimental.pallas import tpu as pltpu

_VMEM_LIMIT_BYTES = 48 * 1024 * 1024
_KERNEL_NUM = 5
_KERNEL_MUL = 2.0


def _stats_body(x_ref, csum_ref, norms_ref, ssq_ref):
    @pl.when(pl.program_id(0) == 0)
    def _init():
        csum_ref[...] = jnp.zeros_like(csum_ref)
        ssq_ref[...] = jnp.zeros_like(ssq_ref)

    x = x_ref[...]
    sq = x * x
    norms = jnp.sum(sq, axis=1, keepdims=True)
    csum_ref[...] = csum_ref[...] + jnp.sum(x, axis=0, keepdims=True)
    norms_ref[...] = norms
    ssq_ref[...] = ssq_ref[...] + jnp.sum(norms)


def _pair_body(ii_ref, jj_ref, coef_ref, sc_ref, m_ref,
               x_ref, y_ref, nr_ref, nc_ref, out_ref, *, chunked):
    s = pl.program_id(0)

    @pl.when(s == 0)
    def _init():
        out_ref[...] = jnp.zeros_like(out_ref)

    t_c = y_ref.shape[0]
    nrs = nr_ref[...] * m_ref[0]

    widths = ((t_c // 8, t_c // 8, t_c // 4, t_c // 2) if chunked else (t_c,))
    tile_sum = jnp.float32(0.0)
    start = 0
    for width in widths:
        yk = y_ref[pl.ds(start, width), :] * sc_ref[0]
        g = jax.lax.dot_general(
            x_ref[...], yk, (((1,), (1,)), ((), ())),
            preferred_element_type=jnp.float32)
        z = g + nrs + nc_ref[:, pl.ds(start, width)] * m_ref[0]
        e = jnp.exp2(z)
        e2 = e * e
        e4 = e2 * e2
        e8 = e4 * e4
        e16 = e8 * e8
        tile_sum = tile_sum + jnp.sum(((e + e2) + (e4 + e8)) + e16)
        start += width

    out_ref[...] = out_ref[...] + coef_ref[s] * tile_sum


def _stats(x, tile):
    b, d = x.shape
    return pl.pallas_call(
        _stats_body,
        out_shape=(jax.ShapeDtypeStruct((1, d), jnp.float32),
                   jax.ShapeDtypeStruct((b, 1), jnp.float32),
                   jax.ShapeDtypeStruct((8, 128), jnp.float32)),
        grid=(b // tile,),
        in_specs=[pl.BlockSpec((tile, d), lambda i: (i, 0))],
        out_specs=(pl.BlockSpec((1, d), lambda i: (0, 0)),
                   pl.BlockSpec((tile, 1), lambda i: (i, 0)),
                   pl.BlockSpec((8, 128), lambda i: (0, 0))),
        compiler_params=pltpu.CompilerParams(
            dimension_semantics=("arbitrary",),
            vmem_limit_bytes=_VMEM_LIMIT_BYTES),
    )(x)


def _pair_sweep(x, y, nr, nc, ii_list, jj_list, coef_list, sc, m,
                tile_r, tile_c, chunked=False):
    d = x.shape[1]
    ns = len(ii_list)
    ii = jnp.asarray(ii_list, dtype=jnp.int32)
    jj = jnp.asarray(jj_list, dtype=jnp.int32)
    coef = jnp.asarray(coef_list, dtype=jnp.float32)
    partials = pl.pallas_call(
        functools.partial(_pair_body, chunked=chunked),
        out_shape=jax.ShapeDtypeStruct((8, 128), jnp.float32),
        grid_spec=pltpu.PrefetchScalarGridSpec(
            num_scalar_prefetch=5,
            grid=(ns,),
            in_specs=[
                pl.BlockSpec((tile_r, d),
                             lambda s, ir, jr, cf, sr, mr: (ir[s], 0)),
                pl.BlockSpec((tile_c, d),
                             lambda s, ir, jr, cf, sr, mr: (jr[s], 0)),
                pl.BlockSpec((tile_r, 1),
                             lambda s, ir, jr, cf, sr, mr: (ir[s], 0)),
                pl.BlockSpec((1, tile_c),
                             lambda s, ir, jr, cf, sr, mr: (0, jr[s])),
            ],
            out_specs=pl.BlockSpec((8, 128),
                                   lambda s, ir, jr, cf, sr, mr: (0, 0)),
        ),
        compiler_params=pltpu.CompilerParams(
            dimension_semantics=("arbitrary",),
            vmem_limit_bytes=_VMEM_LIMIT_BYTES),
    )(ii, jj, coef, sc, m, x, y, nr, nc)
    return partials[0, 0]


def _upper_tri(nt, w2):
    ii, jj, coef = [], [], []
    for a in range(nt):
        for b in range(a, nt):
            ii.append(a)
            jj.append(b)
            coef.append((2.0 if b > a else 1.0) * w2)
    return ii, jj, coef


def _rect(nt_r, nt_c, w2):
    ii, jj, coef = [], [], []
    for a in range(nt_r):
        for b in range(nt_c):
            ii.append(a)
            jj.append(b)
            coef.append(w2)
    return ii, jj, coef


def _mmd_loss(source, target, tile):
    b_src, d = source.shape
    b_tgt = target.shape[0]
    n = b_src + b_tgt
    assert b_src % tile == 0 and b_tgt % tile == 0 and d % 128 == 0
    nt_s = b_src // tile
    nt_t = b_tgt // tile

    source = source.astype(jnp.float32)
    target = target.astype(jnp.float32)

    csum_s, norms_s, ssq_s = _stats(source, tile)
    csum_t, norms_t, ssq_t = _stats(target, tile)

    ssq = ssq_s[0, 0] + ssq_t[0, 0]
    csum = csum_s + csum_t
    sum_l2 = 2.0 * float(n) * ssq - 2.0 * jnp.sum(csum * csum)
    bandwidth = sum_l2 / float(n * n - n)
    bandwidth = bandwidth / (_KERNEL_MUL ** (_KERNEL_NUM // 2))
    inv_bw_largest = 1.0 / (bandwidth * _KERNEL_MUL ** (_KERNEL_NUM - 1))

    log2e = 1.4426950408889634
    sc = (2.0 * log2e * inv_bw_largest).reshape(1).astype(jnp.float32)
    m = (-log2e * inv_bw_largest).reshape(1).astype(jnp.float32)

    nc_s = norms_s.reshape(1, b_src)
    nc_t = norms_t.reshape(1, b_tgt)

    w_ss = 1.0 / (float(b_src) * b_src)
    w_tt = 1.0 / (float(b_tgt) * b_tgt)
    w_st = -2.0 / (float(b_src) * b_tgt)

    tile_st = 2 * tile if (b_src % (2 * tile) == 0 and
                           b_tgt % (2 * tile) == 0) else tile

    p_ss = _pair_sweep(source, source, norms_s, nc_s,
                       *_upper_tri(nt_s, w_ss), sc, m, tile, tile)
    p_tt = _pair_sweep(target, target, norms_t, nc_t,
                       *_upper_tri(nt_t, w_tt), sc, m, tile, tile)
    p_st = _pair_sweep(source, target, norms_s, nc_t,
                       *_rect(b_src // tile_st, b_tgt // tile_st, w_st),
                       sc, m, tile_st, tile_st, chunked=tile_st > tile)

    return p_ss + p_tt + p_st


def kernel(source, target):
    return _mmd_loss(source, target, tile=1024)
